# baseline (speedup 1.0000x reference)
import functools

import jax
import jax.numpy as jnp
from jax import lax
from jax.experimental import pallas as pl
from jax.experimental.pallas import tpu as pltpu
from jax.experimental.pallas import tpu_sc as plsc

N = 50000
K = 32
L = 16
GROUPS = N // L
NW = 32
GPW = -(-GROUPS // NW)
CH = 56
NCHUNK = -(-GPW // CH)
CHL = CH * L
CPT = -(-GROUPS // 16)


def _body(x_hbm, iw_hbm, out_hbm,
          table_v, ids_v, w_v, out_v, part_v, acc_v, shared, sem):
    cid = lax.axis_index("c")
    sid = lax.axis_index("s")
    wid = cid * 16 + sid

    pltpu.sync_copy(x_hbm, table_v)

    lo = sid * CPT
    hi = jnp.minimum(lo + CPT, GROUPS)

    def mean_body(i, acc):
        return acc + table_v[pl.ds(i * L, L)]

    acc = lax.fori_loop(lo, hi, mean_body, jnp.zeros((L,), jnp.float32))
    part_v[...] = acc
    pltpu.sync_copy(part_v, shared.at[cid, sid])
    plsc.subcore_barrier()
    pltpu.sync_copy(shared.at[cid], acc_v)
    tot = jnp.zeros((L,), jnp.float32)
    for j in range(16):
        tot = tot + acc_v[j]
    iota16 = lax.broadcasted_iota(jnp.int32, (L,), 0)
    for s in (1, 2, 4, 8):
        part_v[...] = tot
        tot = tot + plsc.load_gather(part_v, [(iota16 + s) & 15])
    m = tot * (1.0 / N)

    def make_group_body():
        def group_body(gl, g0c):
            off = gl * L
            sw0 = jnp.zeros((L,), jnp.float32)
            sw1 = jnp.zeros((L,), jnp.float32)
            swx0 = jnp.zeros((L,), jnp.float32)
            swx1 = jnp.zeros((L,), jnp.float32)
            swxx0 = jnp.zeros((L,), jnp.float32)
            swxx1 = jnp.zeros((L,), jnp.float32)
            for k in range(0, K, 2):
                w0 = plsc.bitcast(w_v[pl.ds(k * CHL + off, L)], jnp.float32)
                w1 = plsc.bitcast(w_v[pl.ds((k + 1) * CHL + off, L)], jnp.float32)
                nid0 = ids_v[pl.ds(k * CHL + off, L)]
                nid1 = ids_v[pl.ds((k + 1) * CHL + off, L)]
                xg0 = plsc.load_gather(table_v, [nid0])
                xg1 = plsc.load_gather(table_v, [nid1])
                wx0 = w0 * xg0
                wx1 = w1 * xg1
                sw0 = sw0 + w0
                sw1 = sw1 + w1
                swx0 = swx0 + wx0
                swx1 = swx1 + wx1
                swxx0 = swxx0 + wx0 * xg0
                swxx1 = swxx1 + wx1 * xg1
            sw = sw0 + sw1
            swx = swx0 + swx1
            swxx = swxx0 + swxx1
            gg = g0c + gl
            x_vec = table_v[pl.ds(gg * L, L)]
            num = swx - m * sw
            den = swxx - m * (2.0 * swx - m * sw)
            out_v[pl.ds(off, L)] = (x_vec - m) * num * (K - 1.0) / den
            return g0c
        return group_body

    for c in range(NCHUNK):
        g0 = jnp.minimum(wid * GPW + c * CH, GROUPS - CH)
        n0 = g0 * L
        handles = []
        for k in range(K):
            handles.append(pltpu.async_copy(
                iw_hbm.at[pl.ds(k * N + n0, CHL)],
                ids_v.at[pl.ds(k * CHL, CHL)], sem))
            handles.append(pltpu.async_copy(
                iw_hbm.at[pl.ds((K + k) * N + n0, CHL)],
                w_v.at[pl.ds(k * CHL, CHL)], sem))
        for h in handles:
            h.wait()
        plsc.parallel_loop(0, CH, unroll=2, carry=g0)(make_group_body())
        pltpu.sync_copy(out_v, out_hbm.at[pl.ds(n0, CHL)])


@jax.jit
def _moran(x, iw_t):
    mesh = plsc.VectorSubcoreMesh(core_axis_name="c", subcore_axis_name="s")
    return pl.kernel(
        _body,
        out_type=jax.ShapeDtypeStruct((N,), jnp.float32),
        mesh=mesh,
        scratch_types=[
            pltpu.VMEM((N,), jnp.float32),
            pltpu.VMEM((K * CHL,), jnp.int32),
            pltpu.VMEM((K * CHL,), jnp.int32),
            pltpu.VMEM((CHL,), jnp.float32),
            pltpu.VMEM((L,), jnp.float32),
            pltpu.VMEM((16, L), jnp.float32),
            pltpu.HBM((2, 16, L), jnp.float32),
            pltpu.SemaphoreType.DMA,
        ],
        compiler_params=pltpu.CompilerParams(needs_layout_passes=False),
    )(x, iw_t)


def kernel(X, neighbor_weights, neighbor_ids):
    ids_i = neighbor_ids.astype(jnp.int32)
    w_i = jax.lax.bitcast_convert_type(neighbor_weights, jnp.int32)
    iw_t = jnp.stack([ids_i, w_i], axis=0).transpose(0, 2, 1).reshape(-1)
    return _moran(X, iw_t)


# speedup vs baseline: 1.2724x; 1.2724x over previous
import functools

import jax
import jax.numpy as jnp
from jax import lax
from jax.experimental import pallas as pl
from jax.experimental.pallas import tpu as pltpu
from jax.experimental.pallas import tpu_sc as plsc

N = 50000
K = 32
L = 16
GROUPS = N // L
NW = 32
GPW = -(-GROUPS // NW)
CH = 56
NCHUNK = -(-GPW // CH)
CHL = CH * L
CPT = -(-GROUPS // 16)


def _body(x_hbm, ids_hbm, w_hbm, out_hbm,
          table_v, ids_v, w_v, out_v, part_v, acc_v, shared, sem, osem):
    cid = lax.axis_index("c")
    sid = lax.axis_index("s")
    wid = cid * 16 + sid

    pltpu.sync_copy(x_hbm, table_v)

    def fire_chunk(c):
        g0 = jnp.minimum(wid * GPW + c * CH, GROUPS - CH)
        n0 = g0 * L
        hs = []
        for k in range(K):
            hs.append(pltpu.async_copy(
                ids_hbm.at[pl.ds(k * N + n0, CHL)],
                ids_v.at[pl.ds(k * CHL, CHL)], sem))
            hs.append(pltpu.async_copy(
                w_hbm.at[pl.ds(k * N + n0, CHL)],
                w_v.at[pl.ds(k * CHL, CHL)], sem))
        return g0, n0, hs

    g0_0, n0_0, hs_0 = fire_chunk(0)

    lo = sid * CPT
    hi = jnp.minimum(lo + CPT, GROUPS)

    def mean_body(i, acc):
        return acc + table_v[pl.ds(i * L, L)]

    acc = lax.fori_loop(lo, hi, mean_body, jnp.zeros((L,), jnp.float32))
    part_v[...] = acc
    pltpu.sync_copy(part_v, shared.at[cid, sid])
    plsc.subcore_barrier()
    pltpu.sync_copy(shared.at[cid], acc_v)
    tot = jnp.zeros((L,), jnp.float32)
    for j in range(16):
        tot = tot + acc_v[j]
    iota16 = lax.broadcasted_iota(jnp.int32, (L,), 0)
    for s in (1, 2, 4, 8):
        part_v[...] = tot
        tot = tot + plsc.load_gather(part_v, [(iota16 + s) & 15])
    m = tot * (1.0 / N)

    def make_group_body():
        def group_body(gl, g0c):
            off = gl * L
            sw0 = jnp.zeros((L,), jnp.float32)
            sw1 = jnp.zeros((L,), jnp.float32)
            swx0 = jnp.zeros((L,), jnp.float32)
            swx1 = jnp.zeros((L,), jnp.float32)
            swxx0 = jnp.zeros((L,), jnp.float32)
            swxx1 = jnp.zeros((L,), jnp.float32)
            for k in range(0, K, 2):
                w0 = w_v[pl.ds(k * CHL + off, L)]
                w1 = w_v[pl.ds((k + 1) * CHL + off, L)]
                nid0 = ids_v[pl.ds(k * CHL + off, L)]
                nid1 = ids_v[pl.ds((k + 1) * CHL + off, L)]
                xg0 = plsc.load_gather(table_v, [nid0])
                xg1 = plsc.load_gather(table_v, [nid1])
                wx0 = w0 * xg0
                wx1 = w1 * xg1
                sw0 = sw0 + w0
                sw1 = sw1 + w1
                swx0 = swx0 + wx0
                swx1 = swx1 + wx1
                swxx0 = swxx0 + wx0 * xg0
                swxx1 = swxx1 + wx1 * xg1
            sw = sw0 + sw1
            swx = swx0 + swx1
            swxx = swxx0 + swxx1
            gg = g0c + gl
            x_vec = table_v[pl.ds(gg * L, L)]
            num = swx - m * sw
            den = swxx - m * (2.0 * swx - m * sw)
            out_v[pl.ds(off, L)] = (x_vec - m) * num * (K - 1.0) / den
            return g0c
        return group_body

    g0, n0, handles = g0_0, n0_0, hs_0
    out_h = None
    for c in range(NCHUNK):
        for h in handles:
            h.wait()
        if out_h is not None:
            out_h.wait()
        plsc.parallel_loop(0, CH, unroll=2, carry=g0)(make_group_body())
        out_h = pltpu.async_copy(out_v, out_hbm.at[pl.ds(n0, CHL)], osem)
        if c + 1 < NCHUNK:
            g0, n0, handles = fire_chunk(c + 1)
    out_h.wait()


@jax.jit
def _moran(x, ids_t, w_t):
    mesh = plsc.VectorSubcoreMesh(core_axis_name="c", subcore_axis_name="s")
    return pl.kernel(
        _body,
        out_type=jax.ShapeDtypeStruct((N,), jnp.float32),
        mesh=mesh,
        scratch_types=[
            pltpu.VMEM((N,), jnp.float32),
            pltpu.VMEM((K * CHL,), jnp.int32),
            pltpu.VMEM((K * CHL,), jnp.float32),
            pltpu.VMEM((CHL,), jnp.float32),
            pltpu.VMEM((L,), jnp.float32),
            pltpu.VMEM((16, L), jnp.float32),
            pltpu.HBM((2, 16, L), jnp.float32),
            pltpu.SemaphoreType.DMA,
            pltpu.SemaphoreType.DMA,
        ],
        compiler_params=pltpu.CompilerParams(needs_layout_passes=False),
    )(x, ids_t, w_t)


def kernel(X, neighbor_weights, neighbor_ids):
    ids_t = neighbor_ids.astype(jnp.int32).T.reshape(-1)
    w_t = neighbor_weights.T.reshape(-1)
    return _moran(X, ids_t, w_t)


# speedup vs baseline: 1.2908x; 1.0144x over previous
import functools

import jax
import jax.numpy as jnp
from jax import lax
from jax.experimental import pallas as pl
from jax.experimental.pallas import tpu as pltpu
from jax.experimental.pallas import tpu_sc as plsc

N = 50000
K = 32
L = 16
GROUPS = N // L
NW = 32
GPW = -(-GROUPS // NW)
CH = 56
CHUNK_SIZES = (56, 48)
CHL = CH * L
CPT = -(-GROUPS // 16)


def _body(x_hbm, ids_hbm, w_hbm, out_hbm,
          table_v, ids_v, w_v, out_v, part_v, acc_v, shared, sem, osem):
    cid = lax.axis_index("c")
    sid = lax.axis_index("s")
    wid = cid * 16 + sid

    pltpu.sync_copy(x_hbm, table_v)

    def fire_chunk(base, ch):
        g0 = jnp.minimum(base, GROUPS - ch)
        n0 = g0 * L
        chl = ch * L
        hs = []
        for k in range(K):
            hs.append(pltpu.async_copy(
                ids_hbm.at[pl.ds(k * N + n0, chl)],
                ids_v.at[pl.ds(k * CHL, chl)], sem))
            hs.append(pltpu.async_copy(
                w_hbm.at[pl.ds(k * N + n0, chl)],
                w_v.at[pl.ds(k * CHL, chl)], sem))
        return g0, n0, hs

    g0_0, n0_0, hs_0 = fire_chunk(wid * GPW, CHUNK_SIZES[0])

    lo = sid * CPT
    hi = jnp.minimum(lo + CPT, GROUPS)

    def mean_body(i, acc):
        return acc + table_v[pl.ds(i * L, L)]

    acc = lax.fori_loop(lo, hi, mean_body, jnp.zeros((L,), jnp.float32))
    part_v[...] = acc
    pltpu.sync_copy(part_v, shared.at[cid, sid])
    plsc.subcore_barrier()
    pltpu.sync_copy(shared.at[cid], acc_v)
    tot = jnp.zeros((L,), jnp.float32)
    for j in range(16):
        tot = tot + acc_v[j]
    iota16 = lax.broadcasted_iota(jnp.int32, (L,), 0)
    for s in (1, 2, 4, 8):
        part_v[...] = tot
        tot = tot + plsc.load_gather(part_v, [(iota16 + s) & 15])
    m = tot * (1.0 / N)

    def make_group_body():
        def group_body(gl, g0c):
            off = gl * L
            sw0 = jnp.zeros((L,), jnp.float32)
            sw1 = jnp.zeros((L,), jnp.float32)
            swx0 = jnp.zeros((L,), jnp.float32)
            swx1 = jnp.zeros((L,), jnp.float32)
            swxx0 = jnp.zeros((L,), jnp.float32)
            swxx1 = jnp.zeros((L,), jnp.float32)
            for k in range(0, K, 2):
                w0 = w_v[pl.ds(k * CHL + off, L)]
                w1 = w_v[pl.ds((k + 1) * CHL + off, L)]
                nid0 = ids_v[pl.ds(k * CHL + off, L)]
                nid1 = ids_v[pl.ds((k + 1) * CHL + off, L)]
                xg0 = plsc.load_gather(table_v, [nid0])
                xg1 = plsc.load_gather(table_v, [nid1])
                wx0 = w0 * xg0
                wx1 = w1 * xg1
                sw0 = sw0 + w0
                sw1 = sw1 + w1
                swx0 = swx0 + wx0
                swx1 = swx1 + wx1
                swxx0 = swxx0 + wx0 * xg0
                swxx1 = swxx1 + wx1 * xg1
            sw = sw0 + sw1
            swx = swx0 + swx1
            swxx = swxx0 + swxx1
            gg = g0c + gl
            x_vec = table_v[pl.ds(gg * L, L)]
            num = swx - m * sw
            den = swxx - m * (2.0 * swx - m * sw)
            out_v[pl.ds(off, L)] = (x_vec - m) * num * (K - 1.0) / den
            return g0c
        return group_body

    g0, n0, handles = g0_0, n0_0, hs_0
    out_h = None
    for c, ch in enumerate(CHUNK_SIZES):
        for h in handles:
            h.wait()
        if out_h is not None:
            out_h.wait()
        plsc.parallel_loop(0, ch, unroll=2, carry=g0)(make_group_body())
        out_h = pltpu.async_copy(
            out_v.at[pl.ds(0, ch * L)], out_hbm.at[pl.ds(n0, ch * L)], osem)
        if c + 1 < len(CHUNK_SIZES):
            g0, n0, handles = fire_chunk(
                wid * GPW + sum(CHUNK_SIZES[:c + 1]), CHUNK_SIZES[c + 1])
    out_h.wait()


@jax.jit
def _moran(x, ids_t, w_t):
    mesh = plsc.VectorSubcoreMesh(core_axis_name="c", subcore_axis_name="s")
    return pl.kernel(
        _body,
        out_type=jax.ShapeDtypeStruct((N,), jnp.float32),
        mesh=mesh,
        scratch_types=[
            pltpu.VMEM((N,), jnp.float32),
            pltpu.VMEM((K * CHL,), jnp.int32),
            pltpu.VMEM((K * CHL,), jnp.float32),
            pltpu.VMEM((CHL,), jnp.float32),
            pltpu.VMEM((L,), jnp.float32),
            pltpu.VMEM((16, L), jnp.float32),
            pltpu.HBM((2, 16, L), jnp.float32),
            pltpu.SemaphoreType.DMA,
            pltpu.SemaphoreType.DMA,
        ],
        compiler_params=pltpu.CompilerParams(needs_layout_passes=False),
    )(x, ids_t, w_t)


def kernel(X, neighbor_weights, neighbor_ids):
    ids_t = neighbor_ids.astype(jnp.int32).T.reshape(-1)
    w_t = neighbor_weights.T.reshape(-1)
    return _moran(X, ids_t, w_t)
